# baseline (speedup 1.0000x reference)
import functools

import jax
import jax.numpy as jnp
from jax import lax
from jax.experimental import pallas as pl
from jax.experimental.pallas import tpu as pltpu
from jax.experimental.pallas import tpu_sc as plsc

NUM_CORES = 2
NUM_SUBCORES = 16
LANES = 16
NW = NUM_CORES * NUM_SUBCORES
R = 16
NBUF = 4


def _sc_body(D, b_per_w, emb_hbm, idx_hbm, table_hbm, out_hbm,
             table_v, idx_v, ebuf0, ebuf1, ebuf2, ebuf3,
             in_sem0, in_sem1, in_sem2, in_sem3,
             out_sem0, out_sem1, out_sem2, out_sem3):
    c = lax.axis_index("c")
    s = lax.axis_index("s")
    wid = s * NUM_CORES + c
    base = wid * b_per_w
    n_chunks = b_per_w // R

    ebufs = (ebuf0, ebuf1, ebuf2, ebuf3)
    in_sems = (in_sem0, in_sem1, in_sem2, in_sem3)
    out_sems = (out_sem0, out_sem1, out_sem2, out_sem3)

    pltpu.sync_copy(table_hbm, table_v)
    pltpu.sync_copy(idx_hbm.at[wid], idx_v.at[pl.ds(0, b_per_w)])

    def start_in(j, b):
        pltpu.async_copy(emb_hbm.at[pl.ds(base + j * R, R)], ebufs[b],
                         in_sems[b])

    def wait_in(j, b):
        pltpu.make_async_copy(emb_hbm.at[pl.ds(base + j * R, R)], ebufs[b],
                              in_sems[b]).wait()

    def start_out(j, b):
        pltpu.async_copy(ebufs[b], out_hbm.at[pl.ds(base + j * R, R)],
                         out_sems[b])

    def wait_out(j, b):
        pltpu.make_async_copy(ebufs[b], out_hbm.at[pl.ds(base + j * R, R)],
                              out_sems[b]).wait()

    start_in(0, 0)

    def chunk_group(g, carry):
        for bs in range(NBUF):
            j = g + bs

            wait_in(j, bs)

            @pl.when(j + 1 < n_chunks)
            def _prefetch():
                nb = (bs + 1) % NBUF

                @pl.when(j >= NBUF - 1)
                def _free():
                    wait_out(j - (NBUF - 1), nb)

                start_in(j + 1, nb)

            @plsc.parallel_loop(0, R, 1, unroll=4)
            def _add_row(r):
                iv = idx_v[pl.ds(j * R + r, LANES)]
                rbase = iv[0] * D
                for cc in range(D // LANES):
                    v = table_v[pl.ds(rbase + cc * LANES, LANES)]
                    plsc.addupdate(
                        ebufs[bs].at[r, pl.ds(cc * LANES, LANES)], v)

            start_out(j, bs)
        return carry

    lax.fori_loop(0, n_chunks // NBUF, lambda t, cr: chunk_group(t * NBUF, cr),
                  None)

    for j in range(n_chunks - NBUF, n_chunks):
        wait_out(j, j % NBUF)


def kernel(batch_Phrase_emb, Phrase_type_ids, phrase_attribute_emb_all):
    B, D = batch_Phrase_emb.shape
    V = phrase_attribute_emb_all.shape[0]
    b_per_w = B // NW

    idx = Phrase_type_ids.astype(jnp.int32).reshape(NW, b_per_w)
    table_flat = phrase_attribute_emb_all.reshape(V * D)

    mesh = plsc.VectorSubcoreMesh(
        core_axis_name="c", subcore_axis_name="s",
        num_cores=NUM_CORES, num_subcores=NUM_SUBCORES)
    f = pl.kernel(
        functools.partial(_sc_body, D, b_per_w),
        out_type=jax.ShapeDtypeStruct((B, D), jnp.float32),
        mesh=mesh,
        scratch_types=[
            pltpu.VMEM((V * D,), jnp.float32),
            pltpu.VMEM((b_per_w + LANES,), jnp.int32),
            pltpu.VMEM((R, D), jnp.float32),
            pltpu.VMEM((R, D), jnp.float32),
            pltpu.VMEM((R, D), jnp.float32),
            pltpu.VMEM((R, D), jnp.float32),
            pltpu.SemaphoreType.DMA,
            pltpu.SemaphoreType.DMA,
            pltpu.SemaphoreType.DMA,
            pltpu.SemaphoreType.DMA,
            pltpu.SemaphoreType.DMA,
            pltpu.SemaphoreType.DMA,
            pltpu.SemaphoreType.DMA,
            pltpu.SemaphoreType.DMA,
        ],
    )
    return f(batch_Phrase_emb, idx, table_flat)


# speedup vs baseline: 1.4025x; 1.4025x over previous
import functools

import jax
import jax.numpy as jnp
from jax import lax
from jax.experimental import pallas as pl
from jax.experimental.pallas import tpu as pltpu
from jax.experimental.pallas import tpu_sc as plsc

NUM_CORES = 2
NUM_SUBCORES = 16
LANES = 16
NW = NUM_CORES * NUM_SUBCORES
R = 16
NBUF = 4


def _sc_body(D, b_per_w, emb_hbm, idx_hbm, table_hbm, out_hbm,
             table_v, idx_v, ebuf0, ebuf1, ebuf2, ebuf3,
             in_sem0, in_sem1, in_sem2, in_sem3,
             out_sem0, out_sem1, out_sem2, out_sem3):
    c = lax.axis_index("c")
    s = lax.axis_index("s")
    wid = s * NUM_CORES + c
    base = wid * b_per_w
    n_chunks = b_per_w // R

    ebufs = (ebuf0, ebuf1, ebuf2, ebuf3)
    in_sems = (in_sem0, in_sem1, in_sem2, in_sem3)
    out_sems = (out_sem0, out_sem1, out_sem2, out_sem3)

    pltpu.sync_copy(table_hbm, table_v)
    pltpu.sync_copy(idx_hbm.at[wid], idx_v.at[pl.ds(0, b_per_w)])

    def start_in(j, b):
        pltpu.async_copy(emb_hbm.at[pl.ds(base + j * R, R)], ebufs[b],
                         in_sems[b])

    def wait_in(j, b):
        pltpu.make_async_copy(emb_hbm.at[pl.ds(base + j * R, R)], ebufs[b],
                              in_sems[b]).wait()

    def start_out(j, b):
        pltpu.async_copy(ebufs[b], out_hbm.at[pl.ds(base + j * R, R)],
                         out_sems[b])

    def wait_out(j, b):
        pltpu.make_async_copy(ebufs[b], out_hbm.at[pl.ds(base + j * R, R)],
                              out_sems[b]).wait()

    start_in(0, 0)

    def chunk_group(g, carry):
        for bs in range(NBUF):
            j = g + bs

            wait_in(j, bs)

            @pl.when(j + 1 < n_chunks)
            def _prefetch():
                nb = (bs + 1) % NBUF

                @pl.when(j >= NBUF - 1)
                def _free():
                    wait_out(j - (NBUF - 1), nb)

                start_in(j + 1, nb)

            @plsc.parallel_loop(0, R, 1, unroll=1)
            def _add_row(r):
                iv = idx_v[pl.ds(j * R + r, LANES)]
                rbase = iv[0] * D
                for cc in range(D // LANES):
                    v = table_v[pl.ds(rbase + cc * LANES, LANES)]
                    plsc.addupdate(
                        ebufs[bs].at[r, pl.ds(cc * LANES, LANES)], v)

            start_out(j, bs)
        return carry

    lax.fori_loop(0, n_chunks // NBUF, lambda t, cr: chunk_group(t * NBUF, cr),
                  None)

    for j in range(n_chunks - NBUF, n_chunks):
        wait_out(j, j % NBUF)


def kernel(batch_Phrase_emb, Phrase_type_ids, phrase_attribute_emb_all):
    B, D = batch_Phrase_emb.shape
    V = phrase_attribute_emb_all.shape[0]
    b_per_w = B // NW

    idx = Phrase_type_ids.astype(jnp.int32).reshape(NW, b_per_w)
    table_flat = phrase_attribute_emb_all.reshape(V * D)

    mesh = plsc.VectorSubcoreMesh(
        core_axis_name="c", subcore_axis_name="s",
        num_cores=NUM_CORES, num_subcores=NUM_SUBCORES)
    f = pl.kernel(
        functools.partial(_sc_body, D, b_per_w),
        out_type=jax.ShapeDtypeStruct((B, D), jnp.float32),
        mesh=mesh,
        scratch_types=[
            pltpu.VMEM((V * D,), jnp.float32),
            pltpu.VMEM((b_per_w + LANES,), jnp.int32),
            pltpu.VMEM((R, D), jnp.float32),
            pltpu.VMEM((R, D), jnp.float32),
            pltpu.VMEM((R, D), jnp.float32),
            pltpu.VMEM((R, D), jnp.float32),
            pltpu.SemaphoreType.DMA,
            pltpu.SemaphoreType.DMA,
            pltpu.SemaphoreType.DMA,
            pltpu.SemaphoreType.DMA,
            pltpu.SemaphoreType.DMA,
            pltpu.SemaphoreType.DMA,
            pltpu.SemaphoreType.DMA,
            pltpu.SemaphoreType.DMA,
        ],
    )
    return f(batch_Phrase_emb, idx, table_flat)
